# baseline (speedup 1.0000x reference)
import functools

import jax
import jax.numpy as jnp
from jax import lax
from jax.experimental import pallas as pl
from jax.experimental.pallas import tpu as pltpu
from jax.experimental.pallas import tpu_sc as plsc

IMAGE_TOKEN_ID = 151655


@functools.lru_cache(maxsize=None)
def _build_sc_kernel(seq_len: int, hidden: int, num_img: int):
    info = plsc.get_sparse_core_info()
    nc, ns, lanes = info.num_cores, info.num_subcores, info.num_lanes
    nw = nc * ns
    assert seq_len % nw == 0
    chunk = seq_len // nw
    assert chunk == lanes
    nchunks = seq_len // lanes

    mesh = plsc.VectorSubcoreMesh(core_axis_name="c", subcore_axis_name="s")

    @functools.partial(
        pl.kernel,
        out_type=jax.ShapeDtypeStruct((seq_len, hidden), jnp.float32),
        mesh=mesh,
        compiler_params=pltpu.CompilerParams(needs_layout_passes=False),
        scratch_types=[
            pltpu.VMEM((seq_len,), jnp.int32),
            pltpu.VMEM((lanes,), jnp.int32),
            pltpu.VMEM((lanes,), jnp.int32),
            pltpu.VMEM((lanes,), jnp.int32),
            pltpu.VMEM((lanes,), jnp.int32),
            pltpu.VMEM((lanes, hidden), jnp.float32),
            pltpu.VMEM((lanes, hidden), jnp.float32),
            pltpu.SemaphoreType.DMA,
            pltpu.SemaphoreType.DMA,
            pltpu.SemaphoreType.DMA,
            pltpu.SemaphoreType.DMA,
        ],
    )
    def sc_kernel(emb_hbm, img_hbm, ids_hbm, out_hbm,
                  ids_v, idx_text_v, idx_img_v, pos_text_v, pos_img_v,
                  text_v, img_v, sem_a, sem_b, sem_c, sem_d):
        wid = lax.axis_index("c") * ns + lax.axis_index("s")
        base = wid * chunk

        pltpu.sync_copy(ids_hbm, ids_v)

        nbefore = jnp.zeros((lanes,), jnp.int32)
        my_ids = jnp.zeros((lanes,), jnp.int32)
        for j in range(nchunks):
            v = ids_v[pl.ds(j * lanes, lanes)]
            cnt = plsc.all_reduce_population_count(v == IMAGE_TOKEN_ID)
            nbefore = nbefore + jnp.where(j < wid, cnt, 0)
            my_ids = jnp.where(jnp.int32(j) == wid, v, my_ids)

        mask = my_ids == IMAGE_TOKEN_ID
        n_img = jnp.max(plsc.all_reduce_population_count(mask))
        intra = plsc.cumsum(mask.astype(jnp.int32))
        ordinal = jnp.clip(nbefore + intra - 1, 0, num_img - 1)
        lane = lax.iota(jnp.int32, lanes)
        pos = base + lane

        half = lanes // 2

        def piped_copy(src_hbm, idx_ref, buf):
            ga = pltpu.async_copy(
                src_hbm.at[idx_ref.at[pl.ds(0, half)]], buf.at[pl.ds(0, half)], sem_a)
            gb = pltpu.async_copy(
                src_hbm.at[idx_ref.at[pl.ds(half, half)]], buf.at[pl.ds(half, half)], sem_b)
            ga.wait()
            sa = pltpu.async_copy(
                buf.at[pl.ds(0, half)], out_hbm.at[pl.ds(base, half)], sem_c)
            gb.wait()
            sb = pltpu.async_copy(
                buf.at[pl.ds(half, half)], out_hbm.at[pl.ds(base + half, half)], sem_d)
            sa.wait()
            sb.wait()

        @pl.when(n_img == 0)
        def _():
            idx_text_v[...] = my_ids
            piped_copy(emb_hbm, idx_text_v, text_v)

        @pl.when(n_img == lanes)
        def _():
            idx_img_v[...] = ordinal
            piped_copy(img_hbm, idx_img_v, img_v)

        @pl.when(jnp.logical_and(n_img > 0, n_img < lanes))
        def _():
            ft = jnp.min(jnp.where(mask, lanes, lane))
            fi = jnp.min(jnp.where(mask, lane, lanes))
            ftv = jnp.zeros((lanes,), jnp.int32) + ft
            first_text_id = plsc.load_gather(ids_v, [base + ftv])
            idx_text_v[...] = jnp.where(mask, first_text_id, my_ids)
            idx_img_v[...] = jnp.where(mask, ordinal, jnp.clip(nbefore, 0, num_img - 1))
            pos_text_v[...] = jnp.where(mask, base + ft, pos)
            pos_img_v[...] = jnp.where(mask, pos, base + fi)
            cp_t = pltpu.async_copy(emb_hbm.at[idx_text_v], text_v, sem_a)
            cp_i = pltpu.async_copy(img_hbm.at[idx_img_v], img_v, sem_b)
            cp_t.wait()
            st_t = pltpu.async_copy(text_v, out_hbm.at[pos_text_v], sem_a)
            cp_i.wait()
            st_i = pltpu.async_copy(img_v, out_hbm.at[pos_img_v], sem_b)
            st_t.wait()
            st_i.wait()

    return sc_kernel


def kernel(input_ids, image_embeds, embed_weight):
    batch, seq_len = input_ids.shape
    num_img, hidden = image_embeds.shape
    ids = input_ids.reshape(seq_len).astype(jnp.int32)
    sc = _build_sc_kernel(seq_len, hidden, num_img)
    out = sc(embed_weight, image_embeds, ids)
    return out.reshape(batch, seq_len, hidden)


# speedup vs baseline: 1.0518x; 1.0518x over previous
import functools

import jax
import jax.numpy as jnp
from jax import lax
from jax.experimental import pallas as pl
from jax.experimental.pallas import tpu as pltpu
from jax.experimental.pallas import tpu_sc as plsc

IMAGE_TOKEN_ID = 151655


@functools.lru_cache(maxsize=None)
def _build_sc_kernel(seq_len: int, hidden: int, num_img: int):
    info = plsc.get_sparse_core_info()
    nc, ns, lanes = info.num_cores, info.num_subcores, info.num_lanes
    nw = nc * ns
    assert seq_len % nw == 0
    chunk = seq_len // nw
    assert chunk == lanes
    nchunks = seq_len // lanes
    rpw = num_img // nw
    assert rpw * nw == num_img and rpw <= lanes
    assert seq_len - num_img == nw * rpw

    mesh = plsc.VectorSubcoreMesh(core_axis_name="c", subcore_axis_name="s")

    @functools.partial(
        pl.kernel,
        out_type=jax.ShapeDtypeStruct((seq_len, hidden), jnp.float32),
        mesh=mesh,
        compiler_params=pltpu.CompilerParams(needs_layout_passes=False),
        scratch_types=[
            pltpu.VMEM((seq_len,), jnp.int32),
            pltpu.VMEM((rpw,), jnp.int32),
            pltpu.VMEM((rpw,), jnp.int32),
            pltpu.VMEM((rpw,), jnp.int32),
            pltpu.VMEM((lanes,), jnp.int32),
            pltpu.VMEM((lanes,), jnp.int32),
            pltpu.VMEM((lanes,), jnp.int32),
            pltpu.VMEM((lanes,), jnp.int32),
            pltpu.VMEM((lanes, hidden), jnp.float32),
            pltpu.VMEM((lanes, hidden), jnp.float32),
            pltpu.SemaphoreType.DMA,
            pltpu.SemaphoreType.DMA,
            pltpu.SemaphoreType.DMA,
            pltpu.SemaphoreType.DMA,
        ],
    )
    def sc_kernel(emb_hbm, img_hbm, ids_hbm, out_hbm,
                  ids_v, tpos_v, tidx_v, ipos_v,
                  idx_text_v, idx_img_v, pos_text_v, pos_img_v,
                  text_v, img_v, sem_a, sem_b, sem_c, sem_d):
        wid = lax.axis_index("c") * ns + lax.axis_index("s")
        base = wid * chunk
        r0 = wid * rpw

        pltpu.sync_copy(ids_hbm, ids_v)

        lane = lax.iota(jnp.int32, lanes)
        zero = jnp.zeros((lanes,), jnp.int32)

        iprefix = zero
        tprefix = zero
        nbefore = zero
        my_ids = zero
        for j in range(nchunks):
            v = ids_v[pl.ds(j * lanes, lanes)]
            m = v == IMAGE_TOKEN_ID
            ci = plsc.cumsum(m.astype(jnp.int32))
            irank = iprefix + ci - 1
            trank = tprefix + (lane + 1 - ci) - 1
            posj = j * lanes + lane
            til = irank - r0
            sel_i = jnp.logical_and(m, jnp.logical_and(til >= 0, til < rpw))
            plsc.store_scatter(ipos_v, [jnp.clip(til, 0, rpw - 1)], posj,
                               mask=sel_i)
            ttl = trank - r0
            sel_t = jnp.logical_and(jnp.logical_not(m),
                                    jnp.logical_and(ttl >= 0, ttl < rpw))
            ttl_c = jnp.clip(ttl, 0, rpw - 1)
            plsc.store_scatter(tpos_v, [ttl_c], posj, mask=sel_t)
            plsc.store_scatter(tidx_v, [ttl_c], v, mask=sel_t)
            cnt = plsc.all_reduce_population_count(m)
            iprefix = iprefix + cnt
            tprefix = tprefix + (lanes - cnt)
            nbefore = nbefore + jnp.where(j < wid, cnt, 0)
            my_ids = jnp.where(jnp.int32(j) == wid, v, my_ids)

        n_img_tot = jnp.max(iprefix)

        @pl.when(n_img_tot == num_img)
        def _():
            g_txt = pltpu.async_copy(
                emb_hbm.at[tidx_v], text_v.at[pl.ds(0, rpw)], sem_a)
            g_img = pltpu.async_copy(
                img_hbm.at[pl.ds(r0, rpw)], img_v.at[pl.ds(0, rpw)], sem_b)
            g_img.wait()
            s_img = pltpu.async_copy(
                img_v.at[pl.ds(0, rpw)], out_hbm.at[ipos_v], sem_c)
            g_txt.wait()
            s_txt = pltpu.async_copy(
                text_v.at[pl.ds(0, rpw)], out_hbm.at[tpos_v], sem_d)
            s_img.wait()
            s_txt.wait()

        @pl.when(n_img_tot != num_img)
        def _():
            mask = my_ids == IMAGE_TOKEN_ID
            n_img = jnp.max(plsc.all_reduce_population_count(mask))
            intra = plsc.cumsum(mask.astype(jnp.int32))
            ordinal = jnp.clip(nbefore + intra - 1, 0, num_img - 1)
            pos = base + lane

            @pl.when(n_img == 0)
            def _():
                idx_text_v[...] = my_ids
                pltpu.async_copy(emb_hbm.at[idx_text_v], text_v, sem_a).wait()
                pltpu.sync_copy(text_v, out_hbm.at[pl.ds(base, lanes)])

            @pl.when(n_img == lanes)
            def _():
                idx_img_v[...] = ordinal
                pltpu.async_copy(img_hbm.at[idx_img_v], img_v, sem_b).wait()
                pltpu.sync_copy(img_v, out_hbm.at[pl.ds(base, lanes)])

            @pl.when(jnp.logical_and(n_img > 0, n_img < lanes))
            def _():
                ft = jnp.min(jnp.where(mask, lanes, lane))
                fi = jnp.min(jnp.where(mask, lane, lanes))
                ftv = zero + ft
                first_text_id = plsc.load_gather(ids_v, [base + ftv])
                idx_text_v[...] = jnp.where(mask, first_text_id, my_ids)
                idx_img_v[...] = jnp.where(
                    mask, ordinal, jnp.clip(nbefore, 0, num_img - 1))
                pos_text_v[...] = jnp.where(mask, base + ft, pos)
                pos_img_v[...] = jnp.where(mask, pos, base + fi)
                cp_t = pltpu.async_copy(emb_hbm.at[idx_text_v], text_v, sem_a)
                cp_i = pltpu.async_copy(img_hbm.at[idx_img_v], img_v, sem_b)
                cp_t.wait()
                st_t = pltpu.async_copy(text_v, out_hbm.at[pos_text_v], sem_c)
                cp_i.wait()
                st_i = pltpu.async_copy(img_v, out_hbm.at[pos_img_v], sem_d)
                st_t.wait()
                st_i.wait()

    return sc_kernel


def kernel(input_ids, image_embeds, embed_weight):
    batch, seq_len = input_ids.shape
    num_img, hidden = image_embeds.shape
    ids = input_ids.reshape(seq_len).astype(jnp.int32)
    sc = _build_sc_kernel(seq_len, hidden, num_img)
    out = sc(embed_weight, image_embeds, ids)
    return out.reshape(batch, seq_len, hidden)


# speedup vs baseline: 1.0564x; 1.0044x over previous
import functools

import jax
import jax.numpy as jnp
from jax import lax
from jax.experimental import pallas as pl
from jax.experimental.pallas import tpu as pltpu
from jax.experimental.pallas import tpu_sc as plsc

IMAGE_TOKEN_ID = 151655


@functools.lru_cache(maxsize=None)
def _build_sc_kernel(seq_len: int, hidden: int, num_img: int):
    info = plsc.get_sparse_core_info()
    nc, ns, lanes = info.num_cores, info.num_subcores, info.num_lanes
    nw = nc * ns
    assert seq_len % nw == 0
    chunk = seq_len // nw
    assert chunk == lanes
    nchunks = seq_len // lanes
    rpw = num_img // nw
    assert rpw * nw == num_img and rpw <= lanes
    assert seq_len - num_img == nw * rpw

    mesh = plsc.VectorSubcoreMesh(core_axis_name="c", subcore_axis_name="s")

    @functools.partial(
        pl.kernel,
        out_type=jax.ShapeDtypeStruct((seq_len, hidden), jnp.float32),
        mesh=mesh,
        compiler_params=pltpu.CompilerParams(needs_layout_passes=False),
        scratch_types=[
            pltpu.VMEM((seq_len,), jnp.int32),
            pltpu.VMEM((rpw,), jnp.int32),
            pltpu.VMEM((rpw,), jnp.int32),
            pltpu.VMEM((rpw,), jnp.int32),
            pltpu.VMEM((lanes,), jnp.int32),
            pltpu.VMEM((lanes,), jnp.int32),
            pltpu.VMEM((lanes,), jnp.int32),
            pltpu.VMEM((lanes,), jnp.int32),
            pltpu.VMEM((lanes, hidden), jnp.float32),
            pltpu.VMEM((lanes, hidden), jnp.float32),
            pltpu.SemaphoreType.DMA,
            pltpu.SemaphoreType.DMA,
            pltpu.SemaphoreType.DMA,
            pltpu.SemaphoreType.DMA,
        ],
    )
    def sc_kernel(emb_hbm, img_hbm, ids_hbm, out_hbm,
                  ids_v, tpos_v, tidx_v, ipos_v,
                  idx_text_v, idx_img_v, pos_text_v, pos_img_v,
                  text_v, img_v, sem_a, sem_b, sem_c, sem_d):
        wid = lax.axis_index("c") * ns + lax.axis_index("s")
        base = wid * chunk
        r0 = wid * rpw

        g_img = pltpu.async_copy(
            img_hbm.at[pl.ds(r0, rpw)], img_v.at[pl.ds(0, rpw)], sem_b)

        pltpu.sync_copy(ids_hbm, ids_v)

        lane = lax.iota(jnp.int32, lanes)
        zero = jnp.zeros((lanes,), jnp.int32)

        iprefix = zero
        tprefix = zero
        nbefore = zero
        my_ids = zero
        for j in range(nchunks):
            v = ids_v[pl.ds(j * lanes, lanes)]
            m = v == IMAGE_TOKEN_ID
            ci = plsc.cumsum(m.astype(jnp.int32))
            irank = iprefix + ci - 1
            trank = tprefix + (lane + 1 - ci) - 1
            posj = j * lanes + lane
            til = irank - r0
            sel_i = jnp.logical_and(m, jnp.logical_and(til >= 0, til < rpw))
            plsc.store_scatter(ipos_v, [jnp.clip(til, 0, rpw - 1)], posj,
                               mask=sel_i)
            ttl = trank - r0
            sel_t = jnp.logical_and(jnp.logical_not(m),
                                    jnp.logical_and(ttl >= 0, ttl < rpw))
            ttl_c = jnp.clip(ttl, 0, rpw - 1)
            plsc.store_scatter(tpos_v, [ttl_c], posj, mask=sel_t)
            plsc.store_scatter(tidx_v, [ttl_c], v, mask=sel_t)
            cnt = plsc.all_reduce_population_count(m)
            iprefix = iprefix + cnt
            tprefix = tprefix + (lanes - cnt)
            nbefore = nbefore + jnp.where(j < wid, cnt, 0)
            my_ids = jnp.where(jnp.int32(j) == wid, v, my_ids)

        n_img_tot = jnp.max(iprefix)

        @pl.when(n_img_tot == num_img)
        def _():
            g_txt = pltpu.async_copy(
                emb_hbm.at[tidx_v], text_v.at[pl.ds(0, rpw)], sem_a)
            g_img.wait()
            s_img = pltpu.async_copy(
                img_v.at[pl.ds(0, rpw)], out_hbm.at[ipos_v], sem_c)
            g_txt.wait()
            s_txt = pltpu.async_copy(
                text_v.at[pl.ds(0, rpw)], out_hbm.at[tpos_v], sem_d)
            s_img.wait()
            s_txt.wait()

        @pl.when(n_img_tot != num_img)
        def _():
            g_img.wait()
            mask = my_ids == IMAGE_TOKEN_ID
            n_img = jnp.max(plsc.all_reduce_population_count(mask))
            intra = plsc.cumsum(mask.astype(jnp.int32))
            ordinal = jnp.clip(nbefore + intra - 1, 0, num_img - 1)
            pos = base + lane

            @pl.when(n_img == 0)
            def _():
                idx_text_v[...] = my_ids
                pltpu.async_copy(emb_hbm.at[idx_text_v], text_v, sem_a).wait()
                pltpu.sync_copy(text_v, out_hbm.at[pl.ds(base, lanes)])

            @pl.when(n_img == lanes)
            def _():
                idx_img_v[...] = ordinal
                pltpu.async_copy(img_hbm.at[idx_img_v], img_v, sem_b).wait()
                pltpu.sync_copy(img_v, out_hbm.at[pl.ds(base, lanes)])

            @pl.when(jnp.logical_and(n_img > 0, n_img < lanes))
            def _():
                ft = jnp.min(jnp.where(mask, lanes, lane))
                fi = jnp.min(jnp.where(mask, lane, lanes))
                ftv = zero + ft
                first_text_id = plsc.load_gather(ids_v, [base + ftv])
                idx_text_v[...] = jnp.where(mask, first_text_id, my_ids)
                idx_img_v[...] = jnp.where(
                    mask, ordinal, jnp.clip(nbefore, 0, num_img - 1))
                pos_text_v[...] = jnp.where(mask, base + ft, pos)
                pos_img_v[...] = jnp.where(mask, pos, base + fi)
                cp_t = pltpu.async_copy(emb_hbm.at[idx_text_v], text_v, sem_a)
                cp_i = pltpu.async_copy(img_hbm.at[idx_img_v], img_v, sem_b)
                cp_t.wait()
                st_t = pltpu.async_copy(text_v, out_hbm.at[pos_text_v], sem_c)
                cp_i.wait()
                st_i = pltpu.async_copy(img_v, out_hbm.at[pos_img_v], sem_d)
                st_t.wait()
                st_i.wait()

    return sc_kernel


def kernel(input_ids, image_embeds, embed_weight):
    batch, seq_len = input_ids.shape
    num_img, hidden = image_embeds.shape
    ids = input_ids.reshape(seq_len).astype(jnp.int32)
    sc = _build_sc_kernel(seq_len, hidden, num_img)
    out = sc(embed_weight, image_embeds, ids)
    return out.reshape(batch, seq_len, hidden)


# speedup vs baseline: 1.0997x; 1.0409x over previous
import functools

import jax
import jax.numpy as jnp
from jax import lax
from jax.experimental import pallas as pl
from jax.experimental.pallas import tpu as pltpu
from jax.experimental.pallas import tpu_sc as plsc

IMAGE_TOKEN_ID = 151655


@functools.lru_cache(maxsize=None)
def _build_sc_kernel(seq_len: int, hidden: int, num_img: int):
    info = plsc.get_sparse_core_info()
    nc, ns, lanes = info.num_cores, info.num_subcores, info.num_lanes
    nw = nc * ns
    assert seq_len % nw == 0
    chunk = seq_len // nw
    assert chunk == lanes
    nchunks = seq_len // lanes
    rpw = num_img // nw
    assert rpw * nw == num_img and rpw <= lanes
    assert seq_len - num_img == nw * rpw

    mesh = plsc.VectorSubcoreMesh(core_axis_name="c", subcore_axis_name="s")

    @functools.partial(
        pl.kernel,
        out_type=jax.ShapeDtypeStruct((seq_len, hidden), jnp.float32),
        mesh=mesh,
        compiler_params=pltpu.CompilerParams(needs_layout_passes=False),
        scratch_types=[
            pltpu.VMEM((seq_len,), jnp.int32),
            pltpu.VMEM((rpw,), jnp.int32),
            pltpu.VMEM((rpw,), jnp.int32),
            pltpu.VMEM((rpw,), jnp.int32),
            pltpu.VMEM((lanes,), jnp.int32),
            pltpu.VMEM((lanes,), jnp.int32),
            pltpu.VMEM((lanes,), jnp.int32),
            pltpu.VMEM((lanes,), jnp.int32),
            pltpu.VMEM((lanes, hidden), jnp.float32),
            pltpu.VMEM((lanes, hidden), jnp.float32),
            pltpu.SemaphoreType.DMA,
            pltpu.SemaphoreType.DMA,
            pltpu.SemaphoreType.DMA,
            pltpu.SemaphoreType.DMA,
        ],
    )
    def sc_kernel(emb_hbm, img_hbm, ids_hbm, out_hbm,
                  ids_v, tpos_v, tidx_v, ipos_v,
                  idx_text_v, idx_img_v, pos_text_v, pos_img_v,
                  text_v, img_v, sem_a, sem_b, sem_c, sem_d):
        wid = lax.axis_index("c") * ns + lax.axis_index("s")
        base = wid * chunk
        r0 = wid * rpw

        g_img = pltpu.async_copy(
            img_hbm.at[pl.ds(r0, rpw)], img_v.at[pl.ds(0, rpw)], sem_b)

        pltpu.sync_copy(ids_hbm, ids_v)

        lane = lax.iota(jnp.int32, lanes)
        zero = jnp.zeros((lanes,), jnp.int32)

        def scan_body(j, carry):
            iprefix, tprefix, nbefore, my_ids = carry
            v = ids_v[pl.ds(pl.multiple_of(j * lanes, lanes), lanes)]
            m = v == IMAGE_TOKEN_ID
            ci = plsc.cumsum(m.astype(jnp.int32))
            irank = iprefix + ci - 1
            trank = tprefix + (lane + 1 - ci) - 1
            posj = j * lanes + lane
            til = irank - r0
            sel_i = jnp.logical_and(m, jnp.logical_and(til >= 0, til < rpw))
            plsc.store_scatter(ipos_v, [jnp.clip(til, 0, rpw - 1)], posj,
                               mask=sel_i)
            ttl = trank - r0
            sel_t = jnp.logical_and(jnp.logical_not(m),
                                    jnp.logical_and(ttl >= 0, ttl < rpw))
            ttl_c = jnp.clip(ttl, 0, rpw - 1)
            plsc.store_scatter(tpos_v, [ttl_c], posj, mask=sel_t)
            plsc.store_scatter(tidx_v, [ttl_c], v, mask=sel_t)
            cnt = plsc.all_reduce_population_count(m)
            return (iprefix + cnt,
                    tprefix + (lanes - cnt),
                    nbefore + jnp.where(j < wid, cnt, 0),
                    jnp.where(j == wid, v, my_ids))

        iprefix, tprefix, nbefore, my_ids = lax.fori_loop(
            0, nchunks, scan_body, (zero, zero, zero, zero))

        n_img_tot = jnp.max(iprefix)

        @pl.when(n_img_tot == num_img)
        def _():
            g_txt = pltpu.async_copy(
                emb_hbm.at[tidx_v], text_v.at[pl.ds(0, rpw)], sem_a)
            g_img.wait()
            s_img = pltpu.async_copy(
                img_v.at[pl.ds(0, rpw)], out_hbm.at[ipos_v], sem_c)
            g_txt.wait()
            s_txt = pltpu.async_copy(
                text_v.at[pl.ds(0, rpw)], out_hbm.at[tpos_v], sem_d)
            s_img.wait()
            s_txt.wait()

        @pl.when(n_img_tot != num_img)
        def _():
            g_img.wait()
            mask = my_ids == IMAGE_TOKEN_ID
            n_img = jnp.max(plsc.all_reduce_population_count(mask))
            intra = plsc.cumsum(mask.astype(jnp.int32))
            ordinal = jnp.clip(nbefore + intra - 1, 0, num_img - 1)
            pos = base + lane

            @pl.when(n_img == 0)
            def _():
                idx_text_v[...] = my_ids
                pltpu.async_copy(emb_hbm.at[idx_text_v], text_v, sem_a).wait()
                pltpu.sync_copy(text_v, out_hbm.at[pl.ds(base, lanes)])

            @pl.when(n_img == lanes)
            def _():
                idx_img_v[...] = ordinal
                pltpu.async_copy(img_hbm.at[idx_img_v], img_v, sem_b).wait()
                pltpu.sync_copy(img_v, out_hbm.at[pl.ds(base, lanes)])

            @pl.when(jnp.logical_and(n_img > 0, n_img < lanes))
            def _():
                ft = jnp.min(jnp.where(mask, lanes, lane))
                fi = jnp.min(jnp.where(mask, lane, lanes))
                ftv = zero + ft
                first_text_id = plsc.load_gather(ids_v, [base + ftv])
                idx_text_v[...] = jnp.where(mask, first_text_id, my_ids)
                idx_img_v[...] = jnp.where(
                    mask, ordinal, jnp.clip(nbefore, 0, num_img - 1))
                pos_text_v[...] = jnp.where(mask, base + ft, pos)
                pos_img_v[...] = jnp.where(mask, pos, base + fi)
                cp_t = pltpu.async_copy(emb_hbm.at[idx_text_v], text_v, sem_a)
                cp_i = pltpu.async_copy(img_hbm.at[idx_img_v], img_v, sem_b)
                cp_t.wait()
                st_t = pltpu.async_copy(text_v, out_hbm.at[pos_text_v], sem_c)
                cp_i.wait()
                st_i = pltpu.async_copy(img_v, out_hbm.at[pos_img_v], sem_d)
                st_t.wait()
                st_i.wait()

    return sc_kernel


def kernel(input_ids, image_embeds, embed_weight):
    batch, seq_len = input_ids.shape
    num_img, hidden = image_embeds.shape
    ids = input_ids.reshape(seq_len).astype(jnp.int32)
    sc = _build_sc_kernel(seq_len, hidden, num_img)
    out = sc(embed_weight, image_embeds, ids)
    return out.reshape(batch, seq_len, hidden)
